# Initial kernel scaffold; baseline (speedup 1.0000x reference)
import jax
import jax.numpy as jnp
from jax.experimental import pallas as pl


def kernel(x, keys, values, Wq, bq, Wvp, bvp, Wsw, bsw):
    raise NotImplementedError("write your pallas kernel here")


# speedup vs baseline: 1.8131x; 1.8131x over previous
import functools

import jax
import jax.numpy as jnp
from jax import lax
from jax.experimental import pallas as pl
from jax.experimental.pallas import tpu as pltpu
from jax.experimental.pallas import tpu_sc as plsc

INPUT_DIM = 1024
OUTPUT_DIM = 1024
K_DIM = 512
HEADS = 4
KNN = 16
N_KEYS = 256
SIZE = N_KEYS * N_KEYS
V_DIM = OUTPUT_DIM
TOKENS = 2048

TOK_BLK = 256
GRID = TOKENS // TOK_BLK
HALF = K_DIM // 2

_NEG = float("-inf")
_BIG = 1 << 30


def _dotg(a, b):
    return lax.dot_general(a, b, (((1,), (1,)), ((), ())),
                           preferred_element_type=jnp.float32)


def _top16(s, payload=None):
    R = s.shape[0]
    iota = lax.broadcasted_iota(jnp.int32, s.shape, 1)
    iota16 = lax.broadcasted_iota(jnp.int32, (R, KNN), 1)

    def body(i, carry):
        cur, vals, idxs = carry
        m = jnp.max(cur, axis=1, keepdims=True)
        ismax = cur == m
        pos = jnp.min(jnp.where(ismax, iota, _BIG), axis=1, keepdims=True)
        hit = iota == pos
        if payload is None:
            pick = pos
        else:
            pick = jnp.sum(jnp.where(hit, payload, 0), axis=1, keepdims=True)
        sel = iota16 == i
        vals = jnp.where(sel, m, vals)
        idxs = jnp.where(sel, pick, idxs)
        cur = jnp.where(hit, _NEG, cur)
        return cur, vals, idxs

    _, vals, idxs = lax.fori_loop(
        0, KNN, body,
        (s, jnp.zeros((R, KNN), jnp.float32), jnp.zeros((R, KNN), jnp.int32)))
    return vals, idxs


def _topk_body(x_ref, keys_ref, wq_ref, bq_ref, idx_ref, w_ref):
    x = x_ref[...]
    q = _dotg(x, wq_ref[...]) + bq_ref[...]
    s1 = jnp.concatenate(
        [_dotg(q[:, h * K_DIM:h * K_DIM + HALF], keys_ref[h, 0])
         for h in range(HEADS)], axis=0)
    s2 = jnp.concatenate(
        [_dotg(q[:, h * K_DIM + HALF:(h + 1) * K_DIM], keys_ref[h, 1])
         for h in range(HEADS)], axis=0)
    s1v, i1 = _top16(s1)
    s2v, i2 = _top16(s2)
    comb_v = jnp.concatenate([s1v[:, j:j + 1] + s2v for j in range(KNN)],
                             axis=1)
    comb_i = jnp.concatenate(
        [i1[:, j:j + 1] * N_KEYS + i2 for j in range(KNN)], axis=1)
    cv, ci = _top16(comb_v, payload=comb_i)
    m = jnp.max(cv, axis=1, keepdims=True)
    e = jnp.exp(cv - m)
    sm = e / jnp.sum(e, axis=1, keepdims=True)
    B = x.shape[0]
    idx_ref[...] = jnp.concatenate(
        [ci[h * B:(h + 1) * B] for h in range(HEADS)], axis=1)
    w_ref[...] = jnp.concatenate(
        [sm[h * B:(h + 1) * B] for h in range(HEADS)], axis=1)


def _score_topk(x, keys4, Wq, bq):
    return pl.pallas_call(
        _topk_body,
        grid=(GRID,),
        in_specs=[
            pl.BlockSpec((TOK_BLK, INPUT_DIM), lambda i: (i, 0)),
            pl.BlockSpec((HEADS, 2, N_KEYS, HALF), lambda i: (0, 0, 0, 0)),
            pl.BlockSpec((HEADS * K_DIM, INPUT_DIM), lambda i: (0, 0)),
            pl.BlockSpec((HEADS * K_DIM,), lambda i: (0,)),
        ],
        out_specs=[
            pl.BlockSpec((TOK_BLK, HEADS * KNN), lambda i: (i, 0)),
            pl.BlockSpec((TOK_BLK, HEADS * KNN), lambda i: (i, 0)),
        ],
        out_shape=[
            jax.ShapeDtypeStruct((TOKENS, HEADS * KNN), jnp.int32),
            jax.ShapeDtypeStruct((TOKENS, HEADS * KNN), jnp.float32),
        ],
    )(x, keys4, Wq, bq)


_NC, _NS = 2, 16
_NW = _NC * _NS
_TPW = TOKENS // _NW
_HB = HEADS * KNN // 2


def _embbag(values, idx, w):
    mesh = plsc.VectorSubcoreMesh(core_axis_name="c", subcore_axis_name="s")

    @functools.partial(
        pl.kernel,
        out_type=jax.ShapeDtypeStruct((TOKENS, V_DIM), jnp.float32),
        mesh=mesh,
        scratch_types=[
            pltpu.VMEM((_TPW, HEADS * KNN), jnp.int32),
            pltpu.VMEM((_TPW * HEADS * KNN,), jnp.float32),
            pltpu.VMEM((_HB, V_DIM), jnp.float32),
            pltpu.VMEM((_HB, V_DIM), jnp.float32),
            pltpu.VMEM((V_DIM,), jnp.float32),
            pltpu.SemaphoreType.DMA,
            pltpu.SemaphoreType.DMA,
        ],
    )
    def k(values_hbm, idx_hbm, w_hbm, out_hbm,
          idx_v, w_v, buf_a, buf_b, out_v, sem_a, sem_b):
        wid = lax.axis_index("s") * _NC + lax.axis_index("c")
        base = wid * _TPW
        pltpu.sync_copy(idx_hbm.at[pl.ds(base, _TPW), :], idx_v)
        pltpu.sync_copy(
            w_hbm.at[pl.ds(base * (HEADS * KNN), _TPW * HEADS * KNN)], w_v)

        def gather(tok, half, buf, sem):
            return pltpu.make_async_copy(
                values_hbm.at[idx_v.at[tok, pl.ds(half * _HB, _HB)]],
                buf, sem)

        def accum(buf, tok, half):
            for cc in range(V_DIM // 256):
                def row_body(j, acc, _cc=cc):
                    wbase = tok * (HEADS * KNN) + half * _HB
                    wrow = w_v[pl.ds(wbase + (j // 16) * 16, 16)]
                    wv = lax.gather(
                        wrow, jnp.full((16, 1), j % 16, jnp.int32),
                        lax.GatherDimensionNumbers(
                            offset_dims=(), collapsed_slice_dims=(0,),
                            start_index_map=(0,)),
                        (1,), mode=lax.GatherScatterMode.PROMISE_IN_BOUNDS)
                    return tuple(
                        acc[c] + wv * buf[j, pl.ds(_cc * 256 + c * 16, 16)]
                        for c in range(16))
                acc = lax.fori_loop(
                    0, _HB, row_body,
                    tuple(jnp.zeros((16,), jnp.float32) for _ in range(16)))
                for c in range(16):
                    dst = out_v.at[pl.ds(cc * 256 + c * 16, 16)]
                    if half == 0:
                        dst[...] = acc[c]
                    else:
                        plsc.addupdate(dst, acc[c])

        gather(0, 0, buf_a, sem_a).start()
        gather(0, 1, buf_b, sem_b).start()

        def tok_body(t, _):
            gather(t, 0, buf_a, sem_a).wait()
            accum(buf_a, t, 0)

            @pl.when(t < _TPW - 1)
            def _():
                gather(t + 1, 0, buf_a, sem_a).start()

            gather(t, 1, buf_b, sem_b).wait()
            accum(buf_b, t, 1)

            @pl.when(t < _TPW - 1)
            def _():
                gather(t + 1, 1, buf_b, sem_b).start()

            pltpu.sync_copy(out_v, out_hbm.at[base + t])
            return ()

        lax.fori_loop(0, _TPW, tok_body, ())

    return k(values, idx, w.reshape(-1))


def _proj_body(e_ref, x_ref, wsw_ref, bsw_ref, wvp_ref, bvp_ref, o_ref):
    x = x_ref[...]
    g = _dotg(x, wsw_ref[...]) + bsw_ref[...]
    h = e_ref[...] * (g * jax.nn.sigmoid(g))
    o_ref[...] = _dotg(h, wvp_ref[...]) + bvp_ref[...]


def _gate_proj(emb, x, Wsw, bsw, Wvp, bvp):
    return pl.pallas_call(
        _proj_body,
        grid=(GRID,),
        in_specs=[
            pl.BlockSpec((TOK_BLK, V_DIM), lambda i: (i, 0)),
            pl.BlockSpec((TOK_BLK, INPUT_DIM), lambda i: (i, 0)),
            pl.BlockSpec((V_DIM, INPUT_DIM), lambda i: (0, 0)),
            pl.BlockSpec((V_DIM,), lambda i: (0,)),
            pl.BlockSpec((OUTPUT_DIM, V_DIM), lambda i: (0, 0)),
            pl.BlockSpec((OUTPUT_DIM,), lambda i: (0,)),
        ],
        out_specs=pl.BlockSpec((TOK_BLK, OUTPUT_DIM), lambda i: (i, 0)),
        out_shape=jax.ShapeDtypeStruct((TOKENS, OUTPUT_DIM), jnp.float32),
    )(emb, x, Wsw, bsw, Wvp, bvp)


def kernel(x, keys, values, Wq, bq, Wvp, bvp, Wsw, bsw):
    keys4 = keys.reshape(HEADS, 2, N_KEYS, HALF)
    idx, w = _score_topk(x, keys4, Wq, bq)
    emb = _embbag(values, idx, w)
    return _gate_proj(emb, x, Wsw, bsw, Wvp, bvp)


# speedup vs baseline: 1.8393x; 1.0144x over previous
import functools

import jax
import jax.numpy as jnp
from jax import lax
from jax.experimental import pallas as pl
from jax.experimental.pallas import tpu as pltpu
from jax.experimental.pallas import tpu_sc as plsc

INPUT_DIM = 1024
OUTPUT_DIM = 1024
K_DIM = 512
HEADS = 4
KNN = 16
N_KEYS = 256
SIZE = N_KEYS * N_KEYS
V_DIM = OUTPUT_DIM
TOKENS = 2048

TOK_BLK = 256
GRID = TOKENS // TOK_BLK
HALF = K_DIM // 2

_NEG = float("-inf")
_BIG = 1 << 30


def _dotg(a, b):
    return lax.dot_general(a, b, (((1,), (1,)), ((), ())),
                           preferred_element_type=jnp.float32)


def _top16(s, payload=None):
    R = s.shape[0]
    iota = lax.broadcasted_iota(jnp.int32, s.shape, 1)
    iota16 = lax.broadcasted_iota(jnp.int32, (R, KNN), 1)

    def body(i, carry):
        cur, vals, idxs = carry
        m = jnp.max(cur, axis=1, keepdims=True)
        ismax = cur == m
        pos = jnp.min(jnp.where(ismax, iota, _BIG), axis=1, keepdims=True)
        hit = iota == pos
        if payload is None:
            pick = pos
        else:
            pick = jnp.sum(jnp.where(hit, payload, 0), axis=1, keepdims=True)
        sel = iota16 == i
        vals = jnp.where(sel, m, vals)
        idxs = jnp.where(sel, pick, idxs)
        cur = jnp.where(hit, _NEG, cur)
        return cur, vals, idxs

    _, vals, idxs = lax.fori_loop(
        0, KNN, body,
        (s, jnp.zeros((R, KNN), jnp.float32), jnp.zeros((R, KNN), jnp.int32)))
    return vals, idxs


def _topk_body(x_ref, keys_ref, wq_ref, bq_ref, idx_ref, w_ref):
    x = x_ref[...]
    q = _dotg(x, wq_ref[...]) + bq_ref[...]
    s1 = jnp.concatenate(
        [_dotg(q[:, h * K_DIM:h * K_DIM + HALF], keys_ref[h, 0])
         for h in range(HEADS)], axis=0)
    s2 = jnp.concatenate(
        [_dotg(q[:, h * K_DIM + HALF:(h + 1) * K_DIM], keys_ref[h, 1])
         for h in range(HEADS)], axis=0)
    R = s1.shape[0]
    sv, si = _top16(jnp.concatenate([s1, s2], axis=0))
    s1v, i1 = sv[:R], si[:R]
    s2v, i2 = sv[R:], si[R:]
    cols_v, cols_i = [], []
    for j in range(KNN):
        lmax = max(l + 1 for l in range(KNN) if (j + 1) * (l + 1) <= KNN)
        cols_v.append(s1v[:, j:j + 1] + s2v[:, :lmax])
        cols_i.append(i1[:, j:j + 1] * N_KEYS + i2[:, :lmax])
    pad = 64 - sum(c.shape[1] for c in cols_v)
    cols_v.append(jnp.full((R, pad), _NEG, jnp.float32))
    cols_i.append(jnp.zeros((R, pad), jnp.int32))
    comb_v = jnp.concatenate(cols_v, axis=1)
    comb_i = jnp.concatenate(cols_i, axis=1)
    cv, ci = _top16(comb_v, payload=comb_i)
    m = jnp.max(cv, axis=1, keepdims=True)
    e = jnp.exp(cv - m)
    sm = e / jnp.sum(e, axis=1, keepdims=True)
    B = x.shape[0]
    idx_ref[...] = jnp.concatenate(
        [ci[h * B:(h + 1) * B] for h in range(HEADS)], axis=1)
    w_ref[...] = jnp.concatenate(
        [sm[h * B:(h + 1) * B] for h in range(HEADS)], axis=1)


def _score_topk(x, keys4, Wq, bq):
    return pl.pallas_call(
        _topk_body,
        grid=(GRID,),
        in_specs=[
            pl.BlockSpec((TOK_BLK, INPUT_DIM), lambda i: (i, 0)),
            pl.BlockSpec((HEADS, 2, N_KEYS, HALF), lambda i: (0, 0, 0, 0)),
            pl.BlockSpec((HEADS * K_DIM, INPUT_DIM), lambda i: (0, 0)),
            pl.BlockSpec((HEADS * K_DIM,), lambda i: (0,)),
        ],
        out_specs=[
            pl.BlockSpec((TOK_BLK, HEADS * KNN), lambda i: (i, 0)),
            pl.BlockSpec((TOK_BLK, HEADS * KNN), lambda i: (i, 0)),
        ],
        out_shape=[
            jax.ShapeDtypeStruct((TOKENS, HEADS * KNN), jnp.int32),
            jax.ShapeDtypeStruct((TOKENS, HEADS * KNN), jnp.float32),
        ],
    )(x, keys4, Wq, bq)


_NC, _NS = 2, 16
_NW = _NC * _NS
_TPW = TOKENS // _NW
_HB = HEADS * KNN // 2


def _embbag(values, idx, w):
    mesh = plsc.VectorSubcoreMesh(core_axis_name="c", subcore_axis_name="s")

    @functools.partial(
        pl.kernel,
        out_type=jax.ShapeDtypeStruct((TOKENS, V_DIM), jnp.float32),
        mesh=mesh,
        scratch_types=[
            pltpu.VMEM((_TPW, HEADS * KNN), jnp.int32),
            pltpu.VMEM((_TPW * HEADS * KNN,), jnp.float32),
            pltpu.VMEM((_HB, V_DIM), jnp.float32),
            pltpu.VMEM((_HB, V_DIM), jnp.float32),
            pltpu.VMEM((V_DIM,), jnp.float32),
            pltpu.SemaphoreType.DMA,
            pltpu.SemaphoreType.DMA,
        ],
    )
    def k(values_hbm, idx_hbm, w_hbm, out_hbm,
          idx_v, w_v, buf_a, buf_b, out_v, sem_a, sem_b):
        wid = lax.axis_index("s") * _NC + lax.axis_index("c")
        base = wid * _TPW
        pltpu.sync_copy(idx_hbm.at[pl.ds(base, _TPW), :], idx_v)
        pltpu.sync_copy(
            w_hbm.at[pl.ds(base * (HEADS * KNN), _TPW * HEADS * KNN)], w_v)

        def gather(tok, half, buf, sem):
            return pltpu.make_async_copy(
                values_hbm.at[idx_v.at[tok, pl.ds(half * _HB, _HB)]],
                buf, sem)

        def accum(buf, tok, half):
            for cc in range(V_DIM // 256):
                def row_body(j, acc, _cc=cc):
                    wbase = tok * (HEADS * KNN) + half * _HB
                    wrow = w_v[pl.ds(wbase + (j // 16) * 16, 16)]
                    wv = lax.gather(
                        wrow, jnp.full((16, 1), j % 16, jnp.int32),
                        lax.GatherDimensionNumbers(
                            offset_dims=(), collapsed_slice_dims=(0,),
                            start_index_map=(0,)),
                        (1,), mode=lax.GatherScatterMode.PROMISE_IN_BOUNDS)
                    return tuple(
                        acc[c] + wv * buf[j, pl.ds(_cc * 256 + c * 16, 16)]
                        for c in range(16))
                acc = lax.fori_loop(
                    0, _HB, row_body,
                    tuple(jnp.zeros((16,), jnp.float32) for _ in range(16)))
                for c in range(16):
                    dst = out_v.at[pl.ds(cc * 256 + c * 16, 16)]
                    if half == 0:
                        dst[...] = acc[c]
                    else:
                        plsc.addupdate(dst, acc[c])

        gather(0, 0, buf_a, sem_a).start()
        gather(0, 1, buf_b, sem_b).start()

        def tok_body(t, _):
            gather(t, 0, buf_a, sem_a).wait()
            accum(buf_a, t, 0)

            @pl.when(t < _TPW - 1)
            def _():
                gather(t + 1, 0, buf_a, sem_a).start()

            gather(t, 1, buf_b, sem_b).wait()
            accum(buf_b, t, 1)

            @pl.when(t < _TPW - 1)
            def _():
                gather(t + 1, 1, buf_b, sem_b).start()

            pltpu.sync_copy(out_v, out_hbm.at[base + t])
            return ()

        lax.fori_loop(0, _TPW, tok_body, ())

    return k(values, idx, w.reshape(-1))


def _proj_body(e_ref, x_ref, wsw_ref, bsw_ref, wvp_ref, bvp_ref, o_ref):
    x = x_ref[...]
    g = _dotg(x, wsw_ref[...]) + bsw_ref[...]
    h = e_ref[...] * (g * jax.nn.sigmoid(g))
    o_ref[...] = _dotg(h, wvp_ref[...]) + bvp_ref[...]


def _gate_proj(emb, x, Wsw, bsw, Wvp, bvp):
    return pl.pallas_call(
        _proj_body,
        grid=(GRID,),
        in_specs=[
            pl.BlockSpec((TOK_BLK, V_DIM), lambda i: (i, 0)),
            pl.BlockSpec((TOK_BLK, INPUT_DIM), lambda i: (i, 0)),
            pl.BlockSpec((V_DIM, INPUT_DIM), lambda i: (0, 0)),
            pl.BlockSpec((V_DIM,), lambda i: (0,)),
            pl.BlockSpec((OUTPUT_DIM, V_DIM), lambda i: (0, 0)),
            pl.BlockSpec((OUTPUT_DIM,), lambda i: (0,)),
        ],
        out_specs=pl.BlockSpec((TOK_BLK, OUTPUT_DIM), lambda i: (i, 0)),
        out_shape=jax.ShapeDtypeStruct((TOKENS, OUTPUT_DIM), jnp.float32),
    )(emb, x, Wsw, bsw, Wvp, bvp)


def kernel(x, keys, values, Wq, bq, Wvp, bvp, Wsw, bsw):
    keys4 = keys.reshape(HEADS, 2, N_KEYS, HALF)
    idx, w = _score_topk(x, keys4, Wq, bq)
    emb = _embbag(values, idx, w)
    return _gate_proj(emb, x, Wsw, bsw, Wvp, bvp)


# speedup vs baseline: 3.1261x; 1.6996x over previous
import functools

import jax
import jax.numpy as jnp
from jax import lax
from jax.experimental import pallas as pl
from jax.experimental.pallas import tpu as pltpu
from jax.experimental.pallas import tpu_sc as plsc

INPUT_DIM = 1024
OUTPUT_DIM = 1024
K_DIM = 512
HEADS = 4
KNN = 16
N_KEYS = 256
SIZE = N_KEYS * N_KEYS
V_DIM = OUTPUT_DIM
TOKENS = 2048

TOK_BLK = 256
GRID = TOKENS // TOK_BLK
HALF = K_DIM // 2

_NEG = float("-inf")
_BIG = 1 << 30


def _dotg(a, b):
    return lax.dot_general(a, b, (((1,), (1,)), ((), ())),
                           preferred_element_type=jnp.float32)


def _top16_t(s, payload=None):
    C = s.shape[1]
    iota0 = lax.broadcasted_iota(jnp.int32, s.shape, 0)
    iota16 = lax.broadcasted_iota(jnp.int32, (KNN, C), 0)

    def body(i, carry):
        cur, vals, idxs = carry
        m = jnp.max(cur, axis=0, keepdims=True)
        ismax = cur == m
        pos = jnp.min(jnp.where(ismax, iota0, _BIG), axis=0, keepdims=True)
        hit = iota0 == pos
        if payload is None:
            pick = pos
        else:
            pick = jnp.sum(jnp.where(hit, payload, 0), axis=0, keepdims=True)
        sel = iota16 == i
        vals = jnp.where(sel, m, vals)
        idxs = jnp.where(sel, pick, idxs)
        cur = jnp.where(hit, _NEG, cur)
        return cur, vals, idxs

    _, vals, idxs = lax.fori_loop(
        0, KNN, body,
        (s, jnp.zeros((KNN, C), jnp.float32), jnp.zeros((KNN, C), jnp.int32)))
    return vals, idxs


def _topk_body(x_ref, keys_ref, wq_ref, bq_ref, idx_ref, w_ref):
    x = x_ref[...]
    B = x.shape[0]
    q = _dotg(x, wq_ref[...]) + bq_ref[...]
    s12t = jnp.concatenate(
        [_dotg(keys_ref[h, 0], q[:, h * K_DIM:h * K_DIM + HALF])
         for h in range(HEADS)]
        + [_dotg(keys_ref[h, 1], q[:, h * K_DIM + HALF:(h + 1) * K_DIM])
           for h in range(HEADS)], axis=1)
    sv, si = _top16_t(s12t)
    s1v, i1 = sv[:, :4 * B], si[:, :4 * B]
    s2v, i2 = sv[:, 4 * B:], si[:, 4 * B:]
    rows_v, rows_i = [], []
    for j in range(KNN):
        lmax = max(l + 1 for l in range(KNN) if (j + 1) * (l + 1) <= KNN)
        rows_v.append(s1v[j:j + 1] + s2v[:lmax])
        rows_i.append(i1[j:j + 1] * N_KEYS + i2[:lmax])
    pad = 64 - sum(r.shape[0] for r in rows_v)
    rows_v.append(jnp.full((pad, 4 * B), _NEG, jnp.float32))
    rows_i.append(jnp.zeros((pad, 4 * B), jnp.int32))
    comb_v = jnp.concatenate(rows_v, axis=0)
    comb_i = jnp.concatenate(rows_i, axis=0)
    cv, ci = _top16_t(comb_v, payload=comb_i)
    m = jnp.max(cv, axis=0, keepdims=True)
    e = jnp.exp(cv - m)
    sm = e / jnp.sum(e, axis=0, keepdims=True)
    idx_ref[...] = jnp.concatenate(
        [ci[:, h * B:(h + 1) * B].T for h in range(HEADS)], axis=1)
    w_ref[...] = jnp.concatenate(
        [sm[:, h * B:(h + 1) * B].T for h in range(HEADS)], axis=1)


def _score_topk(x, keys4, Wq, bq):
    return pl.pallas_call(
        _topk_body,
        grid=(GRID,),
        in_specs=[
            pl.BlockSpec((TOK_BLK, INPUT_DIM), lambda i: (i, 0)),
            pl.BlockSpec((HEADS, 2, N_KEYS, HALF), lambda i: (0, 0, 0, 0)),
            pl.BlockSpec((HEADS * K_DIM, INPUT_DIM), lambda i: (0, 0)),
            pl.BlockSpec((HEADS * K_DIM,), lambda i: (0,)),
        ],
        out_specs=[
            pl.BlockSpec((TOK_BLK, HEADS * KNN), lambda i: (i, 0)),
            pl.BlockSpec((TOK_BLK, HEADS * KNN), lambda i: (i, 0)),
        ],
        out_shape=[
            jax.ShapeDtypeStruct((TOKENS, HEADS * KNN), jnp.int32),
            jax.ShapeDtypeStruct((TOKENS, HEADS * KNN), jnp.float32),
        ],
    )(x, keys4, Wq, bq)


_NC, _NS = 2, 16
_NW = _NC * _NS
_TPW = TOKENS // _NW
_HB = HEADS * KNN // 2


def _embbag(values, idx, w):
    mesh = plsc.VectorSubcoreMesh(core_axis_name="c", subcore_axis_name="s")

    @functools.partial(
        pl.kernel,
        out_type=jax.ShapeDtypeStruct((TOKENS, V_DIM), jnp.float32),
        mesh=mesh,
        scratch_types=[
            pltpu.VMEM((_TPW, HEADS * KNN), jnp.int32),
            pltpu.VMEM((_TPW * HEADS * KNN,), jnp.float32),
            pltpu.VMEM((_HB, V_DIM), jnp.float32),
            pltpu.VMEM((_HB, V_DIM), jnp.float32),
            pltpu.VMEM((V_DIM,), jnp.float32),
            pltpu.SemaphoreType.DMA,
            pltpu.SemaphoreType.DMA,
        ],
    )
    def k(values_hbm, idx_hbm, w_hbm, out_hbm,
          idx_v, w_v, buf_a, buf_b, out_v, sem_a, sem_b):
        wid = lax.axis_index("s") * _NC + lax.axis_index("c")
        base = wid * _TPW
        pltpu.sync_copy(idx_hbm.at[pl.ds(base, _TPW), :], idx_v)
        pltpu.sync_copy(
            w_hbm.at[pl.ds(base * (HEADS * KNN), _TPW * HEADS * KNN)], w_v)

        def gather(tok, half, buf, sem):
            return pltpu.make_async_copy(
                values_hbm.at[idx_v.at[tok, pl.ds(half * _HB, _HB)]],
                buf, sem)

        def accum(buf, tok, half):
            for cc in range(V_DIM // 256):
                def row_body(j, acc, _cc=cc):
                    wbase = tok * (HEADS * KNN) + half * _HB
                    wrow = w_v[pl.ds(wbase + (j // 16) * 16, 16)]
                    wv = lax.gather(
                        wrow, jnp.full((16, 1), j % 16, jnp.int32),
                        lax.GatherDimensionNumbers(
                            offset_dims=(), collapsed_slice_dims=(0,),
                            start_index_map=(0,)),
                        (1,), mode=lax.GatherScatterMode.PROMISE_IN_BOUNDS)
                    return tuple(
                        acc[c] + wv * buf[j, pl.ds(_cc * 256 + c * 16, 16)]
                        for c in range(16))
                acc = lax.fori_loop(
                    0, _HB, row_body,
                    tuple(jnp.zeros((16,), jnp.float32) for _ in range(16)))
                for c in range(16):
                    dst = out_v.at[pl.ds(cc * 256 + c * 16, 16)]
                    if half == 0:
                        dst[...] = acc[c]
                    else:
                        plsc.addupdate(dst, acc[c])

        gather(0, 0, buf_a, sem_a).start()
        gather(0, 1, buf_b, sem_b).start()

        def tok_body(t, _):
            gather(t, 0, buf_a, sem_a).wait()
            accum(buf_a, t, 0)

            @pl.when(t < _TPW - 1)
            def _():
                gather(t + 1, 0, buf_a, sem_a).start()

            gather(t, 1, buf_b, sem_b).wait()
            accum(buf_b, t, 1)

            @pl.when(t < _TPW - 1)
            def _():
                gather(t + 1, 1, buf_b, sem_b).start()

            pltpu.sync_copy(out_v, out_hbm.at[base + t])
            return ()

        lax.fori_loop(0, _TPW, tok_body, ())

    return k(values, idx, w.reshape(-1))


def _proj_body(e_ref, x_ref, wsw_ref, bsw_ref, wvp_ref, bvp_ref, o_ref):
    x = x_ref[...]
    g = _dotg(x, wsw_ref[...]) + bsw_ref[...]
    h = e_ref[...] * (g * jax.nn.sigmoid(g))
    o_ref[...] = _dotg(h, wvp_ref[...]) + bvp_ref[...]


def _gate_proj(emb, x, Wsw, bsw, Wvp, bvp):
    return pl.pallas_call(
        _proj_body,
        grid=(GRID,),
        in_specs=[
            pl.BlockSpec((TOK_BLK, V_DIM), lambda i: (i, 0)),
            pl.BlockSpec((TOK_BLK, INPUT_DIM), lambda i: (i, 0)),
            pl.BlockSpec((V_DIM, INPUT_DIM), lambda i: (0, 0)),
            pl.BlockSpec((V_DIM,), lambda i: (0,)),
            pl.BlockSpec((OUTPUT_DIM, V_DIM), lambda i: (0, 0)),
            pl.BlockSpec((OUTPUT_DIM,), lambda i: (0,)),
        ],
        out_specs=pl.BlockSpec((TOK_BLK, OUTPUT_DIM), lambda i: (i, 0)),
        out_shape=jax.ShapeDtypeStruct((TOKENS, OUTPUT_DIM), jnp.float32),
    )(emb, x, Wsw, bsw, Wvp, bvp)


def kernel(x, keys, values, Wq, bq, Wvp, bvp, Wsw, bsw):
    keys4 = keys.reshape(HEADS, 2, N_KEYS, HALF)
    idx, w = _score_topk(x, keys4, Wq, bq)
    emb = _embbag(values, idx, w)
    return _gate_proj(emb, x, Wsw, bsw, Wvp, bvp)


# speedup vs baseline: 3.3140x; 1.0601x over previous
import functools

import jax
import jax.numpy as jnp
from jax import lax
from jax.experimental import pallas as pl
from jax.experimental.pallas import tpu as pltpu
from jax.experimental.pallas import tpu_sc as plsc

INPUT_DIM = 1024
OUTPUT_DIM = 1024
K_DIM = 512
HEADS = 4
KNN = 16
N_KEYS = 256
SIZE = N_KEYS * N_KEYS
V_DIM = OUTPUT_DIM
TOKENS = 2048

TOK_BLK = 256
GRID = TOKENS // TOK_BLK
HALF = K_DIM // 2

_NEG = float("-inf")
_BIG = 1 << 30


def _dotg(a, b):
    return lax.dot_general(a, b, (((1,), (1,)), ((), ())),
                           preferred_element_type=jnp.float32)


def _top16_t(s, payload=None):
    C = s.shape[1]
    iota0 = lax.broadcasted_iota(jnp.int32, s.shape, 0)
    iota16 = lax.broadcasted_iota(jnp.int32, (KNN, C), 0)

    def body(i, carry):
        cur, vals, idxs = carry
        m = jnp.max(cur, axis=0, keepdims=True)
        ismax = cur == m
        pos = jnp.min(jnp.where(ismax, iota0, _BIG), axis=0, keepdims=True)
        hit = iota0 == pos
        if payload is None:
            pick = pos
        else:
            pick = jnp.sum(jnp.where(hit, payload, 0), axis=0, keepdims=True)
        sel = iota16 == i
        vals = jnp.where(sel, m, vals)
        idxs = jnp.where(sel, pick, idxs)
        cur = jnp.where(hit, _NEG, cur)
        return cur, vals, idxs

    _, vals, idxs = lax.fori_loop(
        0, KNN, body,
        (s, jnp.zeros((KNN, C), jnp.float32), jnp.zeros((KNN, C), jnp.int32)),
        unroll=2)
    return vals, idxs


def _topk_body(x_ref, keys_ref, wq_ref, bq_ref, idx_ref, w_ref):
    x = x_ref[...]
    B = x.shape[0]
    q = _dotg(x, wq_ref[...]) + bq_ref[...]
    s12t = jnp.concatenate(
        [_dotg(keys_ref[h, 0], q[:, h * K_DIM:h * K_DIM + HALF])
         for h in range(HEADS)]
        + [_dotg(keys_ref[h, 1], q[:, h * K_DIM + HALF:(h + 1) * K_DIM])
           for h in range(HEADS)], axis=1)
    sv, si = _top16_t(s12t)
    s1v, i1 = sv[:, :4 * B], si[:, :4 * B]
    s2v, i2 = sv[:, 4 * B:], si[:, 4 * B:]
    rows_v, rows_i = [], []
    for j in range(KNN):
        lmax = max(l + 1 for l in range(KNN) if (j + 1) * (l + 1) <= KNN)
        rows_v.append(s1v[j:j + 1] + s2v[:lmax])
        rows_i.append(i1[j:j + 1] * N_KEYS + i2[:lmax])
    pad = 64 - sum(r.shape[0] for r in rows_v)
    rows_v.append(jnp.full((pad, 4 * B), _NEG, jnp.float32))
    rows_i.append(jnp.zeros((pad, 4 * B), jnp.int32))
    comb_v = jnp.concatenate(rows_v, axis=0)
    comb_i = jnp.concatenate(rows_i, axis=0)
    cv, ci = _top16_t(comb_v, payload=comb_i)
    m = jnp.max(cv, axis=0, keepdims=True)
    e = jnp.exp(cv - m)
    sm = e / jnp.sum(e, axis=0, keepdims=True)
    idx_ref[...] = jnp.concatenate(
        [ci[:, h * B:(h + 1) * B].T for h in range(HEADS)], axis=1)
    w_ref[...] = jnp.concatenate(
        [sm[:, h * B:(h + 1) * B].T for h in range(HEADS)], axis=1)


def _score_topk(x, keys4, Wq, bq):
    return pl.pallas_call(
        _topk_body,
        grid=(GRID,),
        in_specs=[
            pl.BlockSpec((TOK_BLK, INPUT_DIM), lambda i: (i, 0)),
            pl.BlockSpec((HEADS, 2, N_KEYS, HALF), lambda i: (0, 0, 0, 0)),
            pl.BlockSpec((HEADS * K_DIM, INPUT_DIM), lambda i: (0, 0)),
            pl.BlockSpec((HEADS * K_DIM,), lambda i: (0,)),
        ],
        out_specs=[
            pl.BlockSpec((TOK_BLK, HEADS * KNN), lambda i: (i, 0)),
            pl.BlockSpec((TOK_BLK, HEADS * KNN), lambda i: (i, 0)),
        ],
        out_shape=[
            jax.ShapeDtypeStruct((TOKENS, HEADS * KNN), jnp.int32),
            jax.ShapeDtypeStruct((TOKENS, HEADS * KNN), jnp.float32),
        ],
    )(x, keys4, Wq, bq)


_NC, _NS = 2, 16
_NW = _NC * _NS
_TPW = TOKENS // _NW
_HB = HEADS * KNN // 2


def _embbag(values, idx, w):
    mesh = plsc.VectorSubcoreMesh(core_axis_name="c", subcore_axis_name="s")

    @functools.partial(
        pl.kernel,
        out_type=jax.ShapeDtypeStruct((TOKENS, V_DIM), jnp.float32),
        mesh=mesh,
        scratch_types=[
            pltpu.VMEM((_TPW, HEADS * KNN), jnp.int32),
            pltpu.VMEM((_TPW * HEADS * KNN,), jnp.float32),
            pltpu.VMEM((_HB, V_DIM), jnp.float32),
            pltpu.VMEM((_HB, V_DIM), jnp.float32),
            pltpu.VMEM((V_DIM,), jnp.float32),
            pltpu.SemaphoreType.DMA,
            pltpu.SemaphoreType.DMA,
        ],
    )
    def k(values_hbm, idx_hbm, w_hbm, out_hbm,
          idx_v, w_v, buf_a, buf_b, out_v, sem_a, sem_b):
        wid = lax.axis_index("s") * _NC + lax.axis_index("c")
        base = wid * _TPW
        pltpu.sync_copy(idx_hbm.at[pl.ds(base, _TPW), :], idx_v)
        pltpu.sync_copy(
            w_hbm.at[pl.ds(base * (HEADS * KNN), _TPW * HEADS * KNN)], w_v)

        def gather(tok, half, buf, sem):
            return pltpu.make_async_copy(
                values_hbm.at[idx_v.at[tok, pl.ds(half * _HB, _HB)]],
                buf, sem)

        def accum(buf, tok, half):
            for cc in range(V_DIM // 256):
                def row_body(j, acc, _cc=cc):
                    wbase = tok * (HEADS * KNN) + half * _HB
                    wrow = w_v[pl.ds(wbase + (j // 16) * 16, 16)]
                    wv = lax.gather(
                        wrow, jnp.full((16, 1), j % 16, jnp.int32),
                        lax.GatherDimensionNumbers(
                            offset_dims=(), collapsed_slice_dims=(0,),
                            start_index_map=(0,)),
                        (1,), mode=lax.GatherScatterMode.PROMISE_IN_BOUNDS)
                    return tuple(
                        acc[c] + wv * buf[j, pl.ds(_cc * 256 + c * 16, 16)]
                        for c in range(16))
                acc = lax.fori_loop(
                    0, _HB, row_body,
                    tuple(jnp.zeros((16,), jnp.float32) for _ in range(16)))
                for c in range(16):
                    dst = out_v.at[pl.ds(cc * 256 + c * 16, 16)]
                    if half == 0:
                        dst[...] = acc[c]
                    else:
                        plsc.addupdate(dst, acc[c])

        gather(0, 0, buf_a, sem_a).start()
        gather(0, 1, buf_b, sem_b).start()

        def tok_body(t, _):
            gather(t, 0, buf_a, sem_a).wait()
            accum(buf_a, t, 0)

            @pl.when(t < _TPW - 1)
            def _():
                gather(t + 1, 0, buf_a, sem_a).start()

            gather(t, 1, buf_b, sem_b).wait()
            accum(buf_b, t, 1)

            @pl.when(t < _TPW - 1)
            def _():
                gather(t + 1, 1, buf_b, sem_b).start()

            pltpu.sync_copy(out_v, out_hbm.at[base + t])
            return ()

        lax.fori_loop(0, _TPW, tok_body, ())

    return k(values, idx, w.reshape(-1))


def _proj_body(e_ref, x_ref, wsw_ref, bsw_ref, wvp_ref, bvp_ref, o_ref):
    x = x_ref[...]
    g = _dotg(x, wsw_ref[...]) + bsw_ref[...]
    h = e_ref[...] * (g * jax.nn.sigmoid(g))
    o_ref[...] = _dotg(h, wvp_ref[...]) + bvp_ref[...]


def _gate_proj(emb, x, Wsw, bsw, Wvp, bvp):
    return pl.pallas_call(
        _proj_body,
        grid=(GRID,),
        in_specs=[
            pl.BlockSpec((TOK_BLK, V_DIM), lambda i: (i, 0)),
            pl.BlockSpec((TOK_BLK, INPUT_DIM), lambda i: (i, 0)),
            pl.BlockSpec((V_DIM, INPUT_DIM), lambda i: (0, 0)),
            pl.BlockSpec((V_DIM,), lambda i: (0,)),
            pl.BlockSpec((OUTPUT_DIM, V_DIM), lambda i: (0, 0)),
            pl.BlockSpec((OUTPUT_DIM,), lambda i: (0,)),
        ],
        out_specs=pl.BlockSpec((TOK_BLK, OUTPUT_DIM), lambda i: (i, 0)),
        out_shape=jax.ShapeDtypeStruct((TOKENS, OUTPUT_DIM), jnp.float32),
    )(emb, x, Wsw, bsw, Wvp, bvp)


def kernel(x, keys, values, Wq, bq, Wvp, bvp, Wsw, bsw):
    keys4 = keys.reshape(HEADS, 2, N_KEYS, HALF)
    idx, w = _score_topk(x, keys4, Wq, bq)
    emb = _embbag(values, idx, w)
    return _gate_proj(emb, x, Wsw, bsw, Wvp, bvp)
